# baseline (speedup 1.0000x reference)
import functools

import jax
import jax.numpy as jnp
from jax import lax
from jax.experimental import pallas as pl
from jax.experimental.pallas import tpu as pltpu
from jax.experimental.pallas import tpu_sc as plsc

VOCAB = 50304
HIDDEN = 1024
N_TOK = 4 * 2048
NC = 2
NS = 16
LANES = 16
NW = NC * NS
TOK_PER_W = N_TOK // NW
CHUNK = 16
N_CHUNK = TOK_PER_W // CHUNK
NBUF = 3
PREFETCH = 2
SLICES_PER_ROW = HIDDEN // LANES

_mesh = plsc.VectorSubcoreMesh(core_axis_name="c", subcore_axis_name="s")


@functools.partial(
    pl.kernel,
    mesh=_mesh,
    out_type=jax.ShapeDtypeStruct((N_TOK, HIDDEN), jnp.float32),
    scratch_types=[
        pltpu.VMEM((TOK_PER_W,), jnp.int32),
        pltpu.VMEM((TOK_PER_W,), jnp.int32),
        pltpu.VMEM((NBUF, CHUNK, HIDDEN), jnp.float32),
        pltpu.VMEM((NBUF, CHUNK, HIDDEN), jnp.float32),
    ]
    + [pltpu.SemaphoreType.DMA] * (2 * NBUF),
)
def _embed_sum(ids_hbm, pos_hbm, wtab_hbm, ptab_hbm, out_hbm,
               ids_v, pids_v, wbuf, pbuf, *sems):
    gsem = sems[:NBUF]
    ssem = sems[NBUF:]
    wid = lax.axis_index("s") * NC + lax.axis_index("c")
    base = wid * TOK_PER_W
    pltpu.sync_copy(ids_hbm.at[pl.ds(base, TOK_PER_W)], ids_v)
    pltpu.sync_copy(pos_hbm.at[pl.ds(base, TOK_PER_W)], pids_v)

    def start_g(ci):
        b = ci % NBUF
        idx = pl.ds(ci * CHUNK, CHUNK)
        cw = pltpu.async_copy(wtab_hbm.at[ids_v.at[idx]], wbuf.at[b], gsem[b])
        cp = pltpu.async_copy(ptab_hbm.at[pids_v.at[idx]], pbuf.at[b], gsem[b])
        return cw, cp

    def start_st(ci):
        b = ci % NBUF
        return pltpu.async_copy(
            wbuf.at[b], out_hbm.at[pl.ds(base + ci * CHUNK, CHUNK)], ssem[b])

    g_h = {ci: start_g(ci) for ci in range(min(PREFETCH, N_CHUNK))}
    st_h = {}
    for ci in range(N_CHUNK):
        b = ci % NBUF
        cw, cp = g_h.pop(ci)
        cw.wait()
        cp.wait()

        def row_body(r, carry, _b=b):
            for j in range(SLICES_PER_ROW):
                sl = pl.ds(j * LANES, LANES)
                plsc.addupdate(wbuf.at[_b, r, sl], pbuf[_b, r, sl])
            return carry

        if ci + PREFETCH < N_CHUNK:
            old = ci + PREFETCH - NBUF
            if old >= 0:
                st_h.pop(old).wait()
            g_h[ci + PREFETCH] = start_g(ci + PREFETCH)

        lax.fori_loop(0, CHUNK, row_body, 0)
        st_h[ci] = start_st(ci)
    for ci in sorted(st_h):
        st_h.pop(ci).wait()


def kernel(input_ids, position_ids, word_embeddings, position_embeddings):
    ids = input_ids.reshape(-1).astype(jnp.int32)
    pos = position_ids.reshape(-1).astype(jnp.int32)
    out = _embed_sum(ids, pos, word_embeddings, position_embeddings)
    return out.reshape(input_ids.shape[0], input_ids.shape[1], HIDDEN)


# speedup vs baseline: 1.0583x; 1.0583x over previous
import functools

import jax
import jax.numpy as jnp
from jax import lax
from jax.experimental import pallas as pl
from jax.experimental.pallas import tpu as pltpu
from jax.experimental.pallas import tpu_sc as plsc

VOCAB = 50304
HIDDEN = 1024
N_TOK = 4 * 2048
NC = 2
NS = 16
LANES = 16
NW = NC * NS
TOK_PER_W = N_TOK // NW
CHUNK = 16
N_CHUNK = TOK_PER_W // CHUNK
NBUF = 3
PREFETCH = 2
SLICES_PER_ROW = HIDDEN // LANES

_mesh = plsc.VectorSubcoreMesh(core_axis_name="c", subcore_axis_name="s")


@functools.partial(
    pl.kernel,
    mesh=_mesh,
    out_type=jax.ShapeDtypeStruct((N_TOK, HIDDEN), jnp.float32),
    scratch_types=[
        pltpu.VMEM((TOK_PER_W,), jnp.int32),
        pltpu.VMEM((TOK_PER_W,), jnp.int32),
        pltpu.VMEM((NBUF, CHUNK, HIDDEN), jnp.float32),
        pltpu.VMEM((NBUF, CHUNK, HIDDEN), jnp.float32),
    ]
    + [pltpu.SemaphoreType.DMA] * (2 * NBUF),
)
def _embed_sum(ids_hbm, pos_hbm, wtab_hbm, ptab_hbm, out_hbm,
               ids_v, pids_v, wbuf, pbuf, *sems):
    gsem = sems[:NBUF]
    ssem = sems[NBUF:]
    wid = lax.axis_index("s") * NC + lax.axis_index("c")
    base = wid * TOK_PER_W
    pltpu.sync_copy(ids_hbm.at[pl.ds(base, TOK_PER_W)], ids_v)
    pltpu.sync_copy(pos_hbm.at[pl.ds(base, TOK_PER_W)], pids_v)

    def start_gw(ci):
        b = ci % NBUF
        idx = pl.ds(ci * CHUNK, CHUNK)
        return pltpu.async_copy(wtab_hbm.at[ids_v.at[idx]], wbuf.at[b], gsem[b])

    def start_gp(ci):
        b = ci % NBUF
        idx = pl.ds(ci * CHUNK, CHUNK)
        return pltpu.async_copy(ptab_hbm.at[pids_v.at[idx]], pbuf.at[b], gsem[b])

    def start_st(ci):
        b = ci % NBUF
        return pltpu.async_copy(
            wbuf.at[b], out_hbm.at[pl.ds(base + ci * CHUNK, CHUNK)], ssem[b])

    g_h = {ci: (start_gw(ci), start_gp(ci)) for ci in range(min(PREFETCH, N_CHUNK))}
    st_h = {}
    for ci in range(N_CHUNK):
        b = ci % NBUF
        cw, cp = g_h.pop(ci)
        cw.wait()
        cp.wait()

        def row_body(r, carry, _b=b):
            for j in range(SLICES_PER_ROW):
                sl = pl.ds(j * LANES, LANES)
                plsc.addupdate(wbuf.at[_b, r, sl], pbuf[_b, r, sl])
            return carry

        prefetch = ci + PREFETCH < N_CHUNK
        if prefetch:
            gp_next = start_gp(ci + PREFETCH)

        lax.fori_loop(0, CHUNK, row_body, 0)

        if prefetch:
            old = ci + PREFETCH - NBUF
            if old >= 0:
                st_h.pop(old).wait()
            g_h[ci + PREFETCH] = (start_gw(ci + PREFETCH), gp_next)
        st_h[ci] = start_st(ci)
    for ci in sorted(st_h):
        st_h.pop(ci).wait()


def kernel(input_ids, position_ids, word_embeddings, position_embeddings):
    ids = input_ids.reshape(-1).astype(jnp.int32)
    pos = position_ids.reshape(-1).astype(jnp.int32)
    out = _embed_sum(ids, pos, word_embeddings, position_embeddings)
    return out.reshape(input_ids.shape[0], input_ids.shape[1], HIDDEN)


# speedup vs baseline: 1.0669x; 1.0081x over previous
import functools

import jax
import jax.numpy as jnp
from jax import lax
from jax.experimental import pallas as pl
from jax.experimental.pallas import tpu as pltpu
from jax.experimental.pallas import tpu_sc as plsc

VOCAB = 50304
HIDDEN = 1024
N_TOK = 4 * 2048
NC = 2
NS = 16
LANES = 16
NW = NC * NS
TOK_PER_W = N_TOK // NW
CHUNK = 16
N_CHUNK = TOK_PER_W // CHUNK
NBUF_W = 4
NBUF_P = 3
PRE_W = 3
PRE_P = 2
SLICES_PER_ROW = HIDDEN // LANES

_mesh = plsc.VectorSubcoreMesh(core_axis_name="c", subcore_axis_name="s")


@functools.partial(
    pl.kernel,
    mesh=_mesh,
    out_type=jax.ShapeDtypeStruct((N_TOK, HIDDEN), jnp.float32),
    scratch_types=[
        pltpu.VMEM((TOK_PER_W,), jnp.int32),
        pltpu.VMEM((TOK_PER_W,), jnp.int32),
        pltpu.VMEM((NBUF_W, CHUNK, HIDDEN), jnp.float32),
        pltpu.VMEM((NBUF_P, CHUNK, HIDDEN), jnp.float32),
    ]
    + [pltpu.SemaphoreType.DMA] * (2 * NBUF_W + NBUF_P),
)
def _embed_sum(ids_hbm, pos_hbm, wtab_hbm, ptab_hbm, out_hbm,
               ids_v, pids_v, wbuf, pbuf, *sems):
    wsem = sems[:NBUF_W]
    psem = sems[NBUF_W:NBUF_W + NBUF_P]
    ssem = sems[NBUF_W + NBUF_P:]
    wid = lax.axis_index("s") * NC + lax.axis_index("c")
    base = wid * TOK_PER_W
    pltpu.sync_copy(ids_hbm.at[pl.ds(base, TOK_PER_W)], ids_v)
    pltpu.sync_copy(pos_hbm.at[pl.ds(base, TOK_PER_W)], pids_v)

    def start_gw(ci):
        b = ci % NBUF_W
        idx = pl.ds(ci * CHUNK, CHUNK)
        return pltpu.async_copy(wtab_hbm.at[ids_v.at[idx]], wbuf.at[b], wsem[b])

    def start_gp(ci):
        b = ci % NBUF_P
        idx = pl.ds(ci * CHUNK, CHUNK)
        return pltpu.async_copy(ptab_hbm.at[pids_v.at[idx]], pbuf.at[b], psem[b])

    def start_st(ci):
        b = ci % NBUF_W
        return pltpu.async_copy(
            wbuf.at[b], out_hbm.at[pl.ds(base + ci * CHUNK, CHUNK)], ssem[b])

    gw_h = {ci: start_gw(ci) for ci in range(min(PRE_W, N_CHUNK))}
    gp_h = {ci: start_gp(ci) for ci in range(min(PRE_P, N_CHUNK))}
    st_h = {}
    for ci in range(N_CHUNK):
        wb = ci % NBUF_W
        pb = ci % NBUF_P
        gw_h.pop(ci).wait()
        gp_h.pop(ci).wait()

        if ci + PRE_P < N_CHUNK:
            gp_h[ci + PRE_P] = start_gp(ci + PRE_P)

        def row_body(r, carry, _wb=wb, _pb=pb):
            for j in range(SLICES_PER_ROW):
                sl = pl.ds(j * LANES, LANES)
                plsc.addupdate(wbuf.at[_wb, r, sl], pbuf[_pb, r, sl])
            return carry

        lax.fori_loop(0, CHUNK, row_body, 0)

        if ci + PRE_W < N_CHUNK:
            old = ci + PRE_W - NBUF_W
            if old >= 0:
                st_h.pop(old).wait()
            gw_h[ci + PRE_W] = start_gw(ci + PRE_W)
        st_h[ci] = start_st(ci)
    for ci in sorted(st_h):
        st_h.pop(ci).wait()


def kernel(input_ids, position_ids, word_embeddings, position_embeddings):
    ids = input_ids.reshape(-1).astype(jnp.int32)
    pos = position_ids.reshape(-1).astype(jnp.int32)
    out = _embed_sum(ids, pos, word_embeddings, position_embeddings)
    return out.reshape(input_ids.shape[0], input_ids.shape[1], HIDDEN)


# speedup vs baseline: 1.0762x; 1.0087x over previous
import functools

import jax
import jax.numpy as jnp
from jax import lax
from jax.experimental import pallas as pl
from jax.experimental.pallas import tpu as pltpu
from jax.experimental.pallas import tpu_sc as plsc

VOCAB = 50304
HIDDEN = 1024
N_TOK = 4 * 2048
NC = 2
NS = 16
LANES = 16
NW = NC * NS
TOK_PER_W = N_TOK // NW
CHUNK = 16
N_CHUNK = TOK_PER_W // CHUNK
NBUF_W = 4
NBUF_P = 3
PRE_W = 3
PRE_P = 2
SLICES_PER_ROW = HIDDEN // LANES

_mesh = plsc.VectorSubcoreMesh(core_axis_name="c", subcore_axis_name="s")


@functools.partial(
    pl.kernel,
    mesh=_mesh,
    out_type=jax.ShapeDtypeStruct((N_TOK, HIDDEN), jnp.float32),
    scratch_types=[
        pltpu.VMEM((TOK_PER_W,), jnp.int32),
        pltpu.VMEM((TOK_PER_W,), jnp.int32),
        pltpu.VMEM((NBUF_W, CHUNK, HIDDEN), jnp.float32),
        pltpu.VMEM((NBUF_P, CHUNK, HIDDEN), jnp.float32),
    ]
    + [pltpu.SemaphoreType.DMA] * (2 * NBUF_W + NBUF_P),
)
def _embed_sum(ids_hbm, pos_hbm, wtab_hbm, ptab_hbm, out_hbm,
               ids_v, pids_v, wbuf, pbuf, *sems):
    wsem = sems[:NBUF_W]
    psem = sems[NBUF_W:NBUF_W + NBUF_P]
    ssem = sems[NBUF_W + NBUF_P:]
    wid = lax.axis_index("s") * NC + lax.axis_index("c")
    base = wid * TOK_PER_W
    c_ids = pltpu.async_copy(ids_hbm.at[pl.ds(base, TOK_PER_W)], ids_v, wsem[0])
    c_pids = pltpu.async_copy(pos_hbm.at[pl.ds(base, TOK_PER_W)], pids_v, psem[0])
    c_ids.wait()
    c_pids.wait()

    def start_gw(ci):
        b = ci % NBUF_W
        idx = pl.ds(ci * CHUNK, CHUNK)
        return pltpu.async_copy(wtab_hbm.at[ids_v.at[idx]], wbuf.at[b], wsem[b])

    def start_gp(ci):
        b = ci % NBUF_P
        idx = pl.ds(ci * CHUNK, CHUNK)
        return pltpu.async_copy(ptab_hbm.at[pids_v.at[idx]], pbuf.at[b], psem[b])

    def start_st(ci):
        b = ci % NBUF_W
        return pltpu.async_copy(
            wbuf.at[b], out_hbm.at[pl.ds(base + ci * CHUNK, CHUNK)], ssem[b])

    gw_h = {ci: start_gw(ci) for ci in range(min(PRE_W, N_CHUNK))}
    gp_h = {ci: start_gp(ci) for ci in range(min(PRE_P, N_CHUNK))}
    st_h = {}
    for ci in range(N_CHUNK):
        wb = ci % NBUF_W
        pb = ci % NBUF_P
        gw_h.pop(ci).wait()
        gp_h.pop(ci).wait()

        if ci + PRE_P < N_CHUNK:
            gp_h[ci + PRE_P] = start_gp(ci + PRE_P)

        def row_body(r, carry, _wb=wb, _pb=pb):
            for j in range(SLICES_PER_ROW):
                sl = pl.ds(j * LANES, LANES)
                plsc.addupdate(wbuf.at[_wb, r, sl], pbuf[_pb, r, sl])
            return carry

        lax.fori_loop(0, CHUNK, row_body, 0)

        if ci + PRE_W < N_CHUNK:
            old = ci + PRE_W - NBUF_W
            if old >= 0:
                st_h.pop(old).wait()
            gw_h[ci + PRE_W] = start_gw(ci + PRE_W)
        st_h[ci] = start_st(ci)
    for ci in sorted(st_h):
        st_h.pop(ci).wait()


def kernel(input_ids, position_ids, word_embeddings, position_embeddings):
    ids = input_ids.reshape(-1).astype(jnp.int32)
    pos = position_ids.reshape(-1).astype(jnp.int32)
    out = _embed_sum(ids, pos, word_embeddings, position_embeddings)
    return out.reshape(input_ids.shape[0], input_ids.shape[1], HIDDEN)
